# baseline (speedup 1.0000x reference)
import functools

import jax
import jax.numpy as jnp
from jax import lax
from jax.experimental import pallas as pl
from jax.experimental.pallas import tpu as pltpu
from jax.experimental.pallas import tpu_sc as plsc

EMBED = 64
NC = 2
NS = 16
NW = NC * NS
BB = 128
NBUF = 4


def _transpose_chunk(rows_v, tbuf, j0, evecs):
    for dj in range(4):
        j = j0 * 4 + dj
        jvec = jnp.full((16,), 0, jnp.int32) + j
        for m in range(4):
            v = rows_v[j, pl.ds(16 * m, 16)]
            ti_vec, r_vec = evecs[m]
            plsc.store_scatter(tbuf, [ti_vec, r_vec, jvec], v)


@jax.jit
def _sc_embed(seqT, table):
    S, B = seqT.shape
    nb = B // BB
    nch = S
    mesh = plsc.VectorSubcoreMesh(core_axis_name="c", subcore_axis_name="s")

    @functools.partial(
        pl.kernel,
        mesh=mesh,
        out_type=jax.ShapeDtypeStruct((S, EMBED // 8, nb, 8, BB), jnp.float32),
        scratch_types=[
            pltpu.VMEM((S, BB), jnp.int32),
            pltpu.VMEM((NBUF, BB, EMBED), jnp.float32),
            pltpu.VMEM((NBUF, EMBED // 8, 8, BB), jnp.float32),
            pltpu.SemaphoreType.DMA,
            pltpu.SemaphoreType.DMA,
        ],
        compiler_params=pltpu.CompilerParams(
            use_tc_tiling_on_sc=False, needs_layout_passes=False
        ),
    )
    def k(seq_hbm, tab_hbm, out_hbm, idx_v, rows_v, tbuf_v, gsem, ssem):
        wid = lax.axis_index("s") * NC + lax.axis_index("c")
        pltpu.sync_copy(seq_hbm.at[:, pl.ds(wid * BB, BB)], idx_v)

        lanes = lax.iota(jnp.int32, 16)
        evecs = []
        for m in range(4):
            e = lanes + 16 * m
            evecs.append((e >> 3, e & 7))

        def start_gather(i, b):
            pltpu.async_copy(tab_hbm.at[idx_v.at[i]], rows_v.at[b], gsem)

        def wait_gather(i, b):
            pltpu.make_async_copy(
                tab_hbm.at[idx_v.at[i]], rows_v.at[b], gsem
            ).wait()

        def start_store(i, b):
            pltpu.async_copy(tbuf_v.at[b], out_hbm.at[i, :, wid], ssem)

        def wait_store(i, b):
            pltpu.make_async_copy(
                tbuf_v.at[b], out_hbm.at[i, :, wid], ssem
            ).wait()

        def transpose(b):
            def tr_body(j0, carry):
                _transpose_chunk(rows_v.at[b], tbuf_v.at[b], j0, evecs)
                return carry

            lax.fori_loop(0, BB // 4, tr_body, 0)

        for b in range(NBUF):
            start_gather(b, b)

        for b in range(NBUF):
            wait_gather(b, b)
            transpose(b)
            start_store(b, b)
            start_gather(b + NBUF, b)

        def group(g, carry):
            for b in range(NBUF):
                i = g * NBUF + b
                wait_gather(i, b)
                wait_store(i - NBUF, b)
                transpose(b)
                start_store(i, b)
                start_gather(i + NBUF, b)
            return carry

        lax.fori_loop(1, nch // NBUF - 1, group, 0)

        for b in range(NBUF):
            i = nch - NBUF + b
            wait_gather(i, b)
            wait_store(i - NBUF, b)
            transpose(b)
            start_store(i, b)

        for b in range(NBUF):
            wait_store(nch - NBUF + b, b)

    return k(seqT, table)


def kernel(sequence, table):
    B, S = sequence.shape
    seqT = sequence.T.astype(jnp.int32)
    out6 = _sc_embed(seqT, table.astype(jnp.float32))
    return out6.transpose(2, 4, 0, 1, 3).reshape(B, S, EMBED)


# speedup vs baseline: 1.2166x; 1.2166x over previous
import functools

import jax
import jax.numpy as jnp
from jax import lax
from jax.experimental import pallas as pl
from jax.experimental.pallas import tpu as pltpu
from jax.experimental.pallas import tpu_sc as plsc

EMBED = 64
NC = 2
NS = 16
NW = NC * NS
BB = 128
NBUF = 4


@jax.jit
def _sc_embed(seqT, table):
    S, B = seqT.shape
    nb = B // BB
    nch = S
    mesh = plsc.VectorSubcoreMesh(core_axis_name="c", subcore_axis_name="s")

    @functools.partial(
        pl.kernel,
        mesh=mesh,
        out_type=jax.ShapeDtypeStruct((S, EMBED // 8, nb, 8, BB), jnp.float32),
        scratch_types=[
            pltpu.VMEM((S, BB), jnp.int32),
            pltpu.VMEM((NBUF, BB, EMBED), jnp.float32),
            pltpu.VMEM((NBUF, EMBED // 8, 8, BB), jnp.float32),
            pltpu.SemaphoreType.DMA,
            pltpu.SemaphoreType.DMA,
        ],
        compiler_params=pltpu.CompilerParams(
            use_tc_tiling_on_sc=False, needs_layout_passes=False
        ),
    )
    def k(seq_hbm, tab_hbm, out_hbm, idx_v, rows_v, tbuf_v, gsem, ssem):
        wid = lax.axis_index("s") * NC + lax.axis_index("c")
        pltpu.sync_copy(seq_hbm.at[:, pl.ds(wid * BB, BB)], idx_v)

        lanes = lax.iota(jnp.int32, 16)
        evecs = []
        for m in range(4):
            e = lanes + 16 * m
            evecs.append((e >> 3, e & 7))

        def start_gather(i, b):
            pltpu.async_copy(tab_hbm.at[idx_v.at[i]], rows_v.at[b], gsem)

        def wait_gather(i, b):
            pltpu.make_async_copy(
                tab_hbm.at[idx_v.at[i]], rows_v.at[b], gsem
            ).wait()

        def start_store(i, b):
            pltpu.async_copy(tbuf_v.at[b], out_hbm.at[i, :, wid], ssem)

        def wait_store(i, b):
            pltpu.make_async_copy(
                tbuf_v.at[b], out_hbm.at[i, :, wid], ssem
            ).wait()

        def transpose(b):
            rows = rows_v.at[b]
            tb = tbuf_v.at[b]

            @plsc.parallel_loop(0, BB, step=1, unroll=8)
            def tr(j):
                jvec = lanes * 0 + j
                for m in range(4):
                    v = rows[j, pl.ds(16 * m, 16)]
                    plsc.store_scatter(tb, [evecs[m][0], evecs[m][1], jvec], v)

        for b in range(NBUF):
            start_gather(b, b)

        for b in range(NBUF):
            wait_gather(b, b)
            transpose(b)
            start_store(b, b)
            start_gather(b + NBUF, b)

        def group(g, carry):
            for b in range(NBUF):
                i = g * NBUF + b
                wait_gather(i, b)
                wait_store(i - NBUF, b)
                transpose(b)
                start_store(i, b)
                start_gather(i + NBUF, b)
            return carry

        lax.fori_loop(1, nch // NBUF - 1, group, 0)

        for b in range(NBUF):
            i = nch - NBUF + b
            wait_gather(i, b)
            wait_store(i - NBUF, b)
            transpose(b)
            start_store(i, b)

        for b in range(NBUF):
            wait_store(nch - NBUF + b, b)

    return k(seqT, table)


def kernel(sequence, table):
    B, S = sequence.shape
    seqT = sequence.T.astype(jnp.int32)
    out6 = _sc_embed(seqT, table.astype(jnp.float32))
    return out6.transpose(2, 4, 0, 1, 3).reshape(B, S, EMBED)
